# Initial kernel scaffold; baseline (speedup 1.0000x reference)
import jax
import jax.numpy as jnp
from jax.experimental import pallas as pl


def kernel(x, edge_index, W1, att_src1, att_dst1, b1, W2, att_src2, att_dst2, b2, W3, att_src3, att_dst3, b3, L1W, L1b, L2W, L2b, L3W, L3b):
    raise NotImplementedError("write your pallas kernel here")


# speedup vs baseline: 7.7133x; 7.7133x over previous
import jax
import jax.numpy as jnp
from jax.experimental import pallas as pl
from jax.experimental.pallas import tpu as pltpu
from jax.experimental.pallas import tpu_sc as plsc

N = 10000
E = 320000
D = 128
NPAD = 10240
NW = 32
NT = 16
EPT = E // NW
CAP2 = 448
CH = 32
NCHS = CAP2 // CH
LPAD = NW * CAP2
ROWS = NPAD // NW
TROWS = 336
CHE = 2000
BLK = 1000
GRID = N // BLK

_f32 = jnp.float32
_i32 = jnp.int32

_SCP = pltpu.CompilerParams(use_tc_tiling_on_sc=False,
                            needs_layout_passes=False)
_SCP_EDGE = _SCP


def _bucket(d16):
    return jax.lax.shift_right_logical(
        jax.lax.shift_right_logical(d16, 6) * 6554, 15)


def _vgather(x, idx):
    return jax.lax.gather(
        x, idx[:, None],
        jax.lax.GatherDimensionNumbers(
            offset_dims=(), collapsed_slice_dims=(0,), start_index_map=(0,)),
        (1,), mode=jax.lax.GatherScatterMode.PROMISE_IN_BOUNDS)


def _runs(keys, i16):
    prev = _vgather(keys, jnp.maximum(i16 - 1, 0))
    first = jnp.logical_or(i16 == 0, keys != prev)
    run = plsc.cummax(jnp.where(first, i16, 0))
    nxt = _vgather(keys, jnp.minimum(i16 + 1, 15))
    last = jnp.logical_or(i16 == 15, keys != nxt)
    return run, last


def _part_body(src_hbm, dst_hbm, psrc, pdst,
               es_v, ed_v, os_v, od_v, off_v):
    c = jax.lax.axis_index("c")
    t = jax.lax.axis_index("s")
    wid = c * NT + t
    i16 = jax.lax.iota(_i32, 16)
    z16i = jnp.zeros((16,), _i32)

    off_v[pl.ds(0, 16)] = i16 * CAP2
    off_v[pl.ds(16, 16)] = (i16 + 16) * CAP2

    def _fill(k, carry):
        os_v[pl.ds(k * 16, 16)] = (i16 + k * 16) & 8191
        od_v[pl.ds(k * 16, 16)] = z16i + 2000000
        return carry
    jax.lax.fori_loop(0, LPAD // 16, _fill, 0)

    def _place_chunk(ch, carry):
        pltpu.sync_copy(src_hbm.at[pl.ds(wid * EPT + ch * CHE, CHE)], es_v)
        pltpu.sync_copy(dst_hbm.at[pl.ds(wid * EPT + ch * CHE, CHE)], ed_v)

        def _place(g, carry2):
            s16 = es_v[pl.ds(g * 16, 16)]
            d16 = ed_v[pl.ds(g * 16, 16)]
            bs, perm = plsc.sort_key_val(_bucket(d16), i16)
            run, last = _runs(bs, i16)
            rank = i16 - run
            ss = _vgather(s16, perm)
            ds = _vgather(d16, perm)
            pos = plsc.load_gather(off_v, [bs]) + rank
            plsc.store_scatter(os_v, [pos], ss)
            plsc.store_scatter(od_v, [pos], ds)
            plsc.addupdate_scatter(off_v, [bs], rank + 1, mask=last)
            return carry2
        jax.lax.fori_loop(0, CHE // 16, _place, 0)
        return carry
    jax.lax.fori_loop(0, EPT // CHE, _place_chunk, 0)

    pltpu.sync_copy(os_v.at[pl.ds(0, LPAD)], psrc.at[wid])
    pltpu.sync_copy(od_v.at[pl.ds(0, LPAD)], pdst.at[wid])


def _partition(src, dst):
    mesh = plsc.VectorSubcoreMesh(core_axis_name="c", subcore_axis_name="s")
    fn = pl.kernel(
        _part_body,
        out_type=(jax.ShapeDtypeStruct((NW, LPAD), _i32),
                  jax.ShapeDtypeStruct((NW, LPAD), _i32)),
        mesh=mesh,
        scratch_types=[
            pltpu.VMEM((CHE,), _i32),
            pltpu.VMEM((CHE,), _i32),
            pltpu.VMEM((LPAD + 192,), _i32),
            pltpu.VMEM((LPAD + 192,), _i32),
            pltpu.VMEM((32,), _i32),
        ],
        compiler_params=_SCP,
    )
    return fn(src, dst)


def _logit_body(asdp_hbm, out_hbm, buf_v, out_v):
    c = jax.lax.axis_index("c")
    t = jax.lax.axis_index("s")
    wid = c * NT + t
    i16 = jax.lax.iota(_i32, 16)
    half = jax.lax.shift_right_logical(i16, 1)
    par = jax.lax.bitwise_and(i16, 1)

    for k in range(ROWS // 40):
        pltpu.sync_copy(asdp_hbm.at[pl.ds(wid * ROWS + k * 40, 40)], buf_v)
        for g in range(5):
            v = plsc.load_gather(buf_v, [half + g * 8, par])
            out_v[pl.ds(k * 80 + g * 16, 16)] = v
    pltpu.sync_copy(out_v, out_hbm.at[pl.ds(wid * 2 * ROWS, 2 * ROWS)])


def _logits_1d(asdp):
    mesh = plsc.VectorSubcoreMesh(core_axis_name="c", subcore_axis_name="s")
    fn = pl.kernel(
        _logit_body,
        out_type=jax.ShapeDtypeStruct((2 * NPAD,), _f32),
        mesh=mesh,
        scratch_types=[
            pltpu.VMEM((40, D), _f32),
            pltpu.VMEM((2 * ROWS,), _f32),
        ],
        compiler_params=_SCP,
    )
    return fn(asdp)


def _edge_body(h_hbm, asd1_hbm, psrc, pdst, acc_out,
               src_v, dst_v, dloc_v, rows_v, w_v, tbl_v, s_v, sinv_v, acc_v,
               sem):
    c = jax.lax.axis_index("c")
    t = jax.lax.axis_index("s")
    wid = c * NT + t
    base_row = wid * ROWS
    i16 = jax.lax.iota(_i32, 16)
    z16f = jnp.zeros((16,), _f32)
    z16i = jnp.zeros((16,), _i32)
    ng = D // 16

    pltpu.sync_copy(asd1_hbm, tbl_v)

    def _za(k, carry):
        plsc.store_scatter(acc_v, [z16i + k // ng, i16 + (k % ng) * 16], z16f)
        return carry
    jax.lax.fori_loop(0, TROWS * ng, _za, 0)

    def _zs(k, carry):
        s_v[pl.ds(k * 16, 16)] = z16f
        return carry
    jax.lax.fori_loop(0, TROWS // 16, _zs, 0)

    def _chunk(q, carry):
        w = q // NCHS
        off = wid * CAP2 + (q % NCHS) * CH
        pltpu.sync_copy(psrc.at[w, pl.ds(off, CH)], src_v)
        pltpu.sync_copy(pdst.at[w, pl.ds(off, CH)], dst_v)
        cp = pltpu.async_copy(h_hbm.at[src_v], rows_v, sem)
        for g in range(CH // 16):
            d16 = dst_v[pl.ds(g * 16, 16)]
            s16 = src_v[pl.ds(g * 16, 16)]
            dl16 = jnp.clip(d16 - base_row, 0, ROWS)
            dloc_v[pl.ds(g * 16, 16)] = dl16
            av = plsc.load_gather(tbl_v, [s16 * 2])
            dc = jnp.clip(d16, 0, NPAD - 1)
            dv = plsc.load_gather(tbl_v, [dc * 2 + 1])
            al = av + dv
            al = jnp.where(al >= 0.0, al, al * 0.2)
            wv = jnp.exp(al)
            w_v[pl.ds(g * 16, 16)] = wv
            ks, perm = plsc.sort_key_val(dl16, i16)
            wsrt = _vgather(wv, perm)
            csum = plsc.cumsum(wsrt)
            run, last = _runs(ks, i16)
            pcs = _vgather(csum, jnp.maximum(run - 1, 0))
            pcs = jnp.where(run == 0, 0.0, pcs)
            plsc.addupdate_scatter(s_v, [ks], csum - pcs, mask=last)
        cp.wait()

        def _accum(e, carry3):
            ei = z16i + e
            ws = plsc.load_gather(w_v, [ei])
            dloc = plsc.load_gather(dloc_v, [ei])
            for j in range(ng):
                col = i16 + j * 16
                r = plsc.load_gather(rows_v, [ei, col])
                plsc.addupdate_scatter(acc_v, [dloc, col], r * ws)
            return carry3
        jax.lax.fori_loop(0, CH, _accum, 0)
        return carry
    jax.lax.fori_loop(0, NW * NCHS, _chunk, 0)

    for ck in range(ROWS // CH):
        for g in range(CH // 16):
            n16 = base_row + ck * CH + g * 16 + i16
            nc = jnp.minimum(n16, N - 1)
            src_v[pl.ds(g * 16, 16)] = nc
        cp = pltpu.async_copy(h_hbm.at[src_v], rows_v, sem)
        for g in range(CH // 16):
            n16 = base_row + ck * CH + g * 16 + i16
            nc = jnp.minimum(n16, N - 1)
            av = plsc.load_gather(tbl_v, [nc * 2])
            dv = plsc.load_gather(tbl_v, [nc * 2 + 1])
            al = av + dv
            al = jnp.where(al >= 0.0, al, al * 0.2)
            ws16 = jnp.exp(al)
            w_v[pl.ds(g * 16, 16)] = ws16
            s16 = s_v[pl.ds(ck * CH + g * 16, 16)]
            sinv_v[pl.ds(g * 16, 16)] = 1.0 / (s16 + ws16 + 1e-16)
        cp.wait()

        def _norm(e, carry2):
            ei = z16i + e
            ws = plsc.load_gather(w_v, [ei])
            si = plsc.load_gather(sinv_v, [ei])
            dloc = z16i + ck * CH + e
            for j in range(ng):
                col = i16 + j * 16
                a = plsc.load_gather(acc_v, [dloc, col])
                r = plsc.load_gather(rows_v, [ei, col])
                plsc.store_scatter(acc_v, [dloc, col], (a + ws * r) * si)
            return carry2
        jax.lax.fori_loop(0, CH, _norm, 0)

    for k in range(ROWS // CH):
        pltpu.sync_copy(acc_v.at[pl.ds(k * CH, CH)],
                        acc_out.at[pl.ds(base_row + k * CH, CH)])


def _edge_pass(h, asd1, psrc, pdst):
    mesh = plsc.VectorSubcoreMesh(core_axis_name="c", subcore_axis_name="s")
    fn = pl.kernel(
        _edge_body,
        out_type=jax.ShapeDtypeStruct((NPAD, D), _f32),
        mesh=mesh,
        scratch_types=[
            pltpu.VMEM((CH,), _i32),
            pltpu.VMEM((CH,), _i32),
            pltpu.VMEM((CH,), _i32),
            pltpu.VMEM((CH, D), _f32),
            pltpu.VMEM((CH,), _f32),
            pltpu.VMEM((2 * NPAD,), _f32),
            pltpu.VMEM((TROWS,), _f32),
            pltpu.VMEM((CH,), _f32),
            pltpu.VMEM((TROWS, D), _f32),
            pltpu.SemaphoreType.DMA,
        ],
        compiler_params=_SCP_EDGE,
    )
    return fn(h, asd1, psrc, pdst)


def _asd_pack(asd):
    blk = asd.shape[0]
    return jnp.concatenate([asd, jnp.zeros((blk, D - 2), _f32)], axis=1)


def _pre_body(x_ref, w_ref, a_ref, h_ref, asd_ref):
    h = jnp.dot(x_ref[...], w_ref[...], preferred_element_type=_f32)
    h_ref[...] = h
    asd = jnp.dot(h, a_ref[...], preferred_element_type=_f32)
    asd_ref[...] = _asd_pack(asd)


def _pre(x, W, Acat):
    return pl.pallas_call(
        _pre_body,
        grid=(GRID,),
        in_specs=[pl.BlockSpec((BLK, D), lambda i: (i, 0)),
                  pl.BlockSpec((D, D), lambda i: (0, 0)),
                  pl.BlockSpec((D, 2), lambda i: (0, 0))],
        out_specs=[pl.BlockSpec((BLK, D), lambda i: (i, 0)),
                   pl.BlockSpec((BLK, D), lambda i: (i, 0))],
        out_shape=[jax.ShapeDtypeStruct((N, D), _f32),
                   jax.ShapeDtypeStruct((NPAD, D), _f32)],
    )(x, W, Acat)


def _mid_body(gat_ref, prev_ref, lw_ref, lb_ref, b_ref, wn_ref, an_ref,
              hn_ref, asdn_ref, y_ref, o_ref):
    res = jnp.dot(prev_ref[...], lw_ref[...], preferred_element_type=_f32)
    o = gat_ref[...] + b_ref[...] + res + lb_ref[...]
    o_ref[...] = o
    y = jnp.where(o > 0.0, o, jnp.exp(o) - 1.0)
    y_ref[...] = y
    hn = jnp.dot(y, wn_ref[...], preferred_element_type=_f32)
    hn_ref[...] = hn
    asdn = jnp.dot(hn, an_ref[...], preferred_element_type=_f32)
    asdn_ref[...] = _asd_pack(asdn)


def _mid(gat, prev, LW, Lb, b, Wn, Acatn):
    return pl.pallas_call(
        _mid_body,
        grid=(GRID,),
        in_specs=[pl.BlockSpec((BLK, D), lambda i: (i, 0)),
                  pl.BlockSpec((BLK, D), lambda i: (i, 0)),
                  pl.BlockSpec((D, D), lambda i: (0, 0)),
                  pl.BlockSpec((1, D), lambda i: (0, 0)),
                  pl.BlockSpec((1, D), lambda i: (0, 0)),
                  pl.BlockSpec((D, D), lambda i: (0, 0)),
                  pl.BlockSpec((D, 2), lambda i: (0, 0))],
        out_specs=[pl.BlockSpec((BLK, D), lambda i: (i, 0)),
                   pl.BlockSpec((BLK, D), lambda i: (i, 0)),
                   pl.BlockSpec((BLK, D), lambda i: (i, 0)),
                   pl.BlockSpec((BLK, D), lambda i: (i, 0))],
        out_shape=[jax.ShapeDtypeStruct((N, D), _f32),
                   jax.ShapeDtypeStruct((NPAD, D), _f32),
                   jax.ShapeDtypeStruct((N, D), _f32),
                   jax.ShapeDtypeStruct((N, D), _f32)],
    )(gat, prev, LW, Lb, b, Wn, Acatn)


def _att_cat(att_src, att_dst, pad_to=None):
    a_s = att_src.reshape(-1, 1)
    a_d = att_dst.reshape(-1, 1)
    cat = jnp.concatenate([a_s, a_d], axis=1)
    if pad_to is not None and cat.shape[0] < pad_to:
        cat = jnp.pad(cat, ((0, pad_to - cat.shape[0]), (0, 0)))
    return cat


def kernel(x, edge_index, W1, att_src1, att_dst1, b1, W2, att_src2, att_dst2,
           b2, W3, att_src3, att_dst3, b3, L1W, L1b, L2W, L2b, L3W, L3b):
    src = edge_index[0]
    dst = edge_index[1]

    Acat1 = _att_cat(att_src1, att_dst1)
    Acat2 = _att_cat(att_src2, att_dst2)
    Acat3 = _att_cat(att_src3, att_dst3, pad_to=D)
    W3p = jnp.pad(W3, ((0, 0), (0, D - W3.shape[1])))
    L3Wp = jnp.pad(L3W, ((0, 0), (0, D - L3W.shape[1])))
    pad3 = D - b3.shape[0]
    b3p = jnp.pad(b3, (0, pad3)).reshape(1, D)
    L3bp = jnp.pad(L3b, (0, pad3)).reshape(1, D)

    psrc, pdst = _partition(src, dst)
    h1, asd1 = _pre(x, W1, Acat1)

    LWs = jnp.stack([L1W, L2W, L3Wp])
    Lbs = jnp.stack([L1b.reshape(1, D), L2b.reshape(1, D), L3bp])
    bs = jnp.stack([b1.reshape(1, D), b2.reshape(1, D), b3p])
    Wns = jnp.stack([W2, W3p, W3p])
    Acats = jnp.stack([Acat2, Acat3, Acat3])

    def _layer(carry, params):
        h, asd, prev = carry
        LW, Lb, b, Wn, Acatn = params
        a1d = _logits_1d(asd)
        gat = _edge_pass(h, a1d, psrc, pdst)[:N]
        hn, asdn, y, o = _mid(gat, prev, LW, Lb, b, Wn, Acatn)
        return (hn, asdn, y), o

    _, os = jax.lax.scan(_layer, (h1, asd1, x), (LWs, Lbs, bs, Wns, Acats))
    return os[2][:, :b3.shape[0]]


# speedup vs baseline: 10.3080x; 1.3364x over previous
import jax
import jax.numpy as jnp
from jax.experimental import pallas as pl
from jax.experimental.pallas import tpu as pltpu
from jax.experimental.pallas import tpu_sc as plsc

N = 10000
E = 320000
D = 128
NPAD = 10240
NW = 32
NT = 16
EPT = E // NW
CAP2 = 448
CH = 64
NCHS = CAP2 // CH
LPAD = NW * CAP2
ROWS = NPAD // NW
TROWS = 336
CHE = 2000
BLK = 1000
GRID = N // BLK

_f32 = jnp.float32
_i32 = jnp.int32

_SCP = pltpu.CompilerParams(use_tc_tiling_on_sc=False,
                            needs_layout_passes=False)
_SCP_EDGE = _SCP


def _bucket(d16):
    return jax.lax.shift_right_logical(
        jax.lax.shift_right_logical(d16, 6) * 6554, 15)


def _vgather(x, idx):
    return jax.lax.gather(
        x, idx[:, None],
        jax.lax.GatherDimensionNumbers(
            offset_dims=(), collapsed_slice_dims=(0,), start_index_map=(0,)),
        (1,), mode=jax.lax.GatherScatterMode.PROMISE_IN_BOUNDS)


def _runs(keys, i16):
    prev = _vgather(keys, jnp.maximum(i16 - 1, 0))
    first = jnp.logical_or(i16 == 0, keys != prev)
    run = plsc.cummax(jnp.where(first, i16, 0))
    nxt = _vgather(keys, jnp.minimum(i16 + 1, 15))
    last = jnp.logical_or(i16 == 15, keys != nxt)
    return run, last


def _part_body(src_hbm, dst_hbm, psrc, pdst,
               es_v, ed_v, os_v, od_v, off_v):
    c = jax.lax.axis_index("c")
    t = jax.lax.axis_index("s")
    wid = c * NT + t
    i16 = jax.lax.iota(_i32, 16)
    z16i = jnp.zeros((16,), _i32)

    off_v[pl.ds(0, 16)] = i16 * CAP2
    off_v[pl.ds(16, 16)] = (i16 + 16) * CAP2

    def _fill(k, carry):
        os_v[pl.ds(k * 16, 16)] = (i16 + k * 16) & 8191
        od_v[pl.ds(k * 16, 16)] = z16i + 2000000
        return carry
    jax.lax.fori_loop(0, LPAD // 16, _fill, 0)

    def _place_chunk(ch, carry):
        pltpu.sync_copy(src_hbm.at[pl.ds(wid * EPT + ch * CHE, CHE)], es_v)
        pltpu.sync_copy(dst_hbm.at[pl.ds(wid * EPT + ch * CHE, CHE)], ed_v)

        def _place(g, carry2):
            s16 = es_v[pl.ds(g * 16, 16)]
            d16 = ed_v[pl.ds(g * 16, 16)]
            bs, perm = plsc.sort_key_val(_bucket(d16), i16)
            run, last = _runs(bs, i16)
            rank = i16 - run
            ss = _vgather(s16, perm)
            ds = _vgather(d16, perm)
            pos = plsc.load_gather(off_v, [bs]) + rank
            plsc.store_scatter(os_v, [pos], ss)
            plsc.store_scatter(od_v, [pos], ds)
            plsc.addupdate_scatter(off_v, [bs], rank + 1, mask=last)
            return carry2
        jax.lax.fori_loop(0, CHE // 16, _place, 0)
        return carry
    jax.lax.fori_loop(0, EPT // CHE, _place_chunk, 0)

    pltpu.sync_copy(os_v.at[pl.ds(0, LPAD)], psrc.at[wid])
    pltpu.sync_copy(od_v.at[pl.ds(0, LPAD)], pdst.at[wid])


def _partition(src, dst):
    mesh = plsc.VectorSubcoreMesh(core_axis_name="c", subcore_axis_name="s")
    fn = pl.kernel(
        _part_body,
        out_type=(jax.ShapeDtypeStruct((NW, LPAD), _i32),
                  jax.ShapeDtypeStruct((NW, LPAD), _i32)),
        mesh=mesh,
        scratch_types=[
            pltpu.VMEM((CHE,), _i32),
            pltpu.VMEM((CHE,), _i32),
            pltpu.VMEM((LPAD + 192,), _i32),
            pltpu.VMEM((LPAD + 192,), _i32),
            pltpu.VMEM((32,), _i32),
        ],
        compiler_params=_SCP,
    )
    return fn(src, dst)


def _logit_body(asdp_hbm, out_hbm, buf_v, out_v):
    c = jax.lax.axis_index("c")
    t = jax.lax.axis_index("s")
    wid = c * NT + t
    i16 = jax.lax.iota(_i32, 16)
    half = jax.lax.shift_right_logical(i16, 1)
    par = jax.lax.bitwise_and(i16, 1)

    for k in range(ROWS // 40):
        pltpu.sync_copy(asdp_hbm.at[pl.ds(wid * ROWS + k * 40, 40)], buf_v)
        for g in range(5):
            v = plsc.load_gather(buf_v, [half + g * 8, par])
            out_v[pl.ds(k * 80 + g * 16, 16)] = v
    pltpu.sync_copy(out_v, out_hbm.at[pl.ds(wid * 2 * ROWS, 2 * ROWS)])


def _logits_1d(asdp):
    mesh = plsc.VectorSubcoreMesh(core_axis_name="c", subcore_axis_name="s")
    fn = pl.kernel(
        _logit_body,
        out_type=jax.ShapeDtypeStruct((2 * NPAD,), _f32),
        mesh=mesh,
        scratch_types=[
            pltpu.VMEM((40, D), _f32),
            pltpu.VMEM((2 * ROWS,), _f32),
        ],
        compiler_params=_SCP,
    )
    return fn(asdp)


def _edge_body(h_hbm, asd1_hbm, psrc, pdst, acc_out,
               src_v, dst_v, dloc_v, rows_v, w_v, tbl_v, s_v, sinv_v, acc_v,
               sem):
    c = jax.lax.axis_index("c")
    t = jax.lax.axis_index("s")
    wid = c * NT + t
    base_row = wid * ROWS
    i16 = jax.lax.iota(_i32, 16)
    z16f = jnp.zeros((16,), _f32)
    z16i = jnp.zeros((16,), _i32)
    ng = D // 16

    pltpu.sync_copy(asd1_hbm, tbl_v)

    def _za(k, carry):
        plsc.store_scatter(acc_v, [z16i + k // ng, i16 + (k % ng) * 16], z16f)
        return carry
    jax.lax.fori_loop(0, TROWS * ng, _za, 0)

    def _zs(k, carry):
        s_v[pl.ds(k * 16, 16)] = z16f
        return carry
    jax.lax.fori_loop(0, TROWS // 16, _zs, 0)

    def _chunk(q, carry):
        w = q // NCHS
        off = wid * CAP2 + (q % NCHS) * CH
        pltpu.sync_copy(psrc.at[w, pl.ds(off, CH)], src_v)
        pltpu.sync_copy(pdst.at[w, pl.ds(off, CH)], dst_v)
        cp = pltpu.async_copy(h_hbm.at[src_v], rows_v, sem)
        for g in range(CH // 16):
            d16 = dst_v[pl.ds(g * 16, 16)]
            s16 = src_v[pl.ds(g * 16, 16)]
            dl16 = jnp.clip(d16 - base_row, 0, ROWS)
            dloc_v[pl.ds(g * 16, 16)] = dl16
            av = plsc.load_gather(tbl_v, [s16 * 2])
            dc = jnp.clip(d16, 0, NPAD - 1)
            dv = plsc.load_gather(tbl_v, [dc * 2 + 1])
            al = av + dv
            al = jnp.where(al >= 0.0, al, al * 0.2)
            wv = jnp.exp(al)
            w_v[pl.ds(g * 16, 16)] = wv
            ks, perm = plsc.sort_key_val(dl16, i16)
            wsrt = _vgather(wv, perm)
            csum = plsc.cumsum(wsrt)
            run, last = _runs(ks, i16)
            pcs = _vgather(csum, jnp.maximum(run - 1, 0))
            pcs = jnp.where(run == 0, 0.0, pcs)
            plsc.addupdate_scatter(s_v, [ks], csum - pcs, mask=last)
        cp.wait()

        def _accum(e, carry3):
            ei = z16i + e
            ws = plsc.load_gather(w_v, [ei])
            dloc = plsc.load_gather(dloc_v, [ei])
            for j in range(ng):
                col = i16 + j * 16
                r = plsc.load_gather(rows_v, [ei, col])
                plsc.addupdate_scatter(acc_v, [dloc, col], r * ws)
            return carry3
        jax.lax.fori_loop(0, CH, _accum, 0, unroll=4)
        return carry
    jax.lax.fori_loop(0, NW * NCHS, _chunk, 0)

    for ck in range(ROWS // CH):
        for g in range(CH // 16):
            n16 = base_row + ck * CH + g * 16 + i16
            nc = jnp.minimum(n16, N - 1)
            src_v[pl.ds(g * 16, 16)] = nc
        cp = pltpu.async_copy(h_hbm.at[src_v], rows_v, sem)
        for g in range(CH // 16):
            n16 = base_row + ck * CH + g * 16 + i16
            nc = jnp.minimum(n16, N - 1)
            av = plsc.load_gather(tbl_v, [nc * 2])
            dv = plsc.load_gather(tbl_v, [nc * 2 + 1])
            al = av + dv
            al = jnp.where(al >= 0.0, al, al * 0.2)
            ws16 = jnp.exp(al)
            w_v[pl.ds(g * 16, 16)] = ws16
            s16 = s_v[pl.ds(ck * CH + g * 16, 16)]
            sinv_v[pl.ds(g * 16, 16)] = 1.0 / (s16 + ws16 + 1e-16)
        cp.wait()

        def _norm(e, carry2):
            ei = z16i + e
            ws = plsc.load_gather(w_v, [ei])
            si = plsc.load_gather(sinv_v, [ei])
            dloc = z16i + ck * CH + e
            for j in range(ng):
                col = i16 + j * 16
                a = plsc.load_gather(acc_v, [dloc, col])
                r = plsc.load_gather(rows_v, [ei, col])
                plsc.store_scatter(acc_v, [dloc, col], (a + ws * r) * si)
            return carry2
        jax.lax.fori_loop(0, CH, _norm, 0)

    for k in range(ROWS // CH):
        pltpu.sync_copy(acc_v.at[pl.ds(k * CH, CH)],
                        acc_out.at[pl.ds(base_row + k * CH, CH)])


def _edge_pass(h, asd1, psrc, pdst):
    mesh = plsc.VectorSubcoreMesh(core_axis_name="c", subcore_axis_name="s")
    fn = pl.kernel(
        _edge_body,
        out_type=jax.ShapeDtypeStruct((NPAD, D), _f32),
        mesh=mesh,
        scratch_types=[
            pltpu.VMEM((CH,), _i32),
            pltpu.VMEM((CH,), _i32),
            pltpu.VMEM((CH,), _i32),
            pltpu.VMEM((CH, D), _f32),
            pltpu.VMEM((CH,), _f32),
            pltpu.VMEM((2 * NPAD,), _f32),
            pltpu.VMEM((TROWS,), _f32),
            pltpu.VMEM((CH,), _f32),
            pltpu.VMEM((TROWS, D), _f32),
            pltpu.SemaphoreType.DMA,
        ],
        compiler_params=_SCP_EDGE,
    )
    return fn(h, asd1, psrc, pdst)


def _asd_pack(asd):
    blk = asd.shape[0]
    return jnp.concatenate([asd, jnp.zeros((blk, D - 2), _f32)], axis=1)


def _pre_body(x_ref, w_ref, a_ref, h_ref, asd_ref):
    h = jnp.dot(x_ref[...], w_ref[...], preferred_element_type=_f32)
    h_ref[...] = h
    asd = jnp.dot(h, a_ref[...], preferred_element_type=_f32)
    asd_ref[...] = _asd_pack(asd)


def _pre(x, W, Acat):
    return pl.pallas_call(
        _pre_body,
        grid=(GRID,),
        in_specs=[pl.BlockSpec((BLK, D), lambda i: (i, 0)),
                  pl.BlockSpec((D, D), lambda i: (0, 0)),
                  pl.BlockSpec((D, 2), lambda i: (0, 0))],
        out_specs=[pl.BlockSpec((BLK, D), lambda i: (i, 0)),
                   pl.BlockSpec((BLK, D), lambda i: (i, 0))],
        out_shape=[jax.ShapeDtypeStruct((N, D), _f32),
                   jax.ShapeDtypeStruct((NPAD, D), _f32)],
    )(x, W, Acat)


def _mid_body(gat_ref, prev_ref, lw_ref, lb_ref, b_ref, wn_ref, an_ref,
              hn_ref, asdn_ref, y_ref, o_ref):
    res = jnp.dot(prev_ref[...], lw_ref[...], preferred_element_type=_f32)
    o = gat_ref[...] + b_ref[...] + res + lb_ref[...]
    o_ref[...] = o
    y = jnp.where(o > 0.0, o, jnp.exp(o) - 1.0)
    y_ref[...] = y
    hn = jnp.dot(y, wn_ref[...], preferred_element_type=_f32)
    hn_ref[...] = hn
    asdn = jnp.dot(hn, an_ref[...], preferred_element_type=_f32)
    asdn_ref[...] = _asd_pack(asdn)


def _mid(gat, prev, LW, Lb, b, Wn, Acatn):
    return pl.pallas_call(
        _mid_body,
        grid=(GRID,),
        in_specs=[pl.BlockSpec((BLK, D), lambda i: (i, 0)),
                  pl.BlockSpec((BLK, D), lambda i: (i, 0)),
                  pl.BlockSpec((D, D), lambda i: (0, 0)),
                  pl.BlockSpec((1, D), lambda i: (0, 0)),
                  pl.BlockSpec((1, D), lambda i: (0, 0)),
                  pl.BlockSpec((D, D), lambda i: (0, 0)),
                  pl.BlockSpec((D, 2), lambda i: (0, 0))],
        out_specs=[pl.BlockSpec((BLK, D), lambda i: (i, 0)),
                   pl.BlockSpec((BLK, D), lambda i: (i, 0)),
                   pl.BlockSpec((BLK, D), lambda i: (i, 0)),
                   pl.BlockSpec((BLK, D), lambda i: (i, 0))],
        out_shape=[jax.ShapeDtypeStruct((N, D), _f32),
                   jax.ShapeDtypeStruct((NPAD, D), _f32),
                   jax.ShapeDtypeStruct((N, D), _f32),
                   jax.ShapeDtypeStruct((N, D), _f32)],
    )(gat, prev, LW, Lb, b, Wn, Acatn)


def _att_cat(att_src, att_dst, pad_to=None):
    a_s = att_src.reshape(-1, 1)
    a_d = att_dst.reshape(-1, 1)
    cat = jnp.concatenate([a_s, a_d], axis=1)
    if pad_to is not None and cat.shape[0] < pad_to:
        cat = jnp.pad(cat, ((0, pad_to - cat.shape[0]), (0, 0)))
    return cat


def kernel(x, edge_index, W1, att_src1, att_dst1, b1, W2, att_src2, att_dst2,
           b2, W3, att_src3, att_dst3, b3, L1W, L1b, L2W, L2b, L3W, L3b):
    src = edge_index[0]
    dst = edge_index[1]

    Acat1 = _att_cat(att_src1, att_dst1)
    Acat2 = _att_cat(att_src2, att_dst2)
    Acat3 = _att_cat(att_src3, att_dst3, pad_to=D)
    W3p = jnp.pad(W3, ((0, 0), (0, D - W3.shape[1])))
    L3Wp = jnp.pad(L3W, ((0, 0), (0, D - L3W.shape[1])))
    pad3 = D - b3.shape[0]
    b3p = jnp.pad(b3, (0, pad3)).reshape(1, D)
    L3bp = jnp.pad(L3b, (0, pad3)).reshape(1, D)

    psrc, pdst = _partition(src, dst)
    h1, asd1 = _pre(x, W1, Acat1)

    LWs = jnp.stack([L1W, L2W, L3Wp])
    Lbs = jnp.stack([L1b.reshape(1, D), L2b.reshape(1, D), L3bp])
    bs = jnp.stack([b1.reshape(1, D), b2.reshape(1, D), b3p])
    Wns = jnp.stack([W2, W3p, W3p])
    Acats = jnp.stack([Acat2, Acat3, Acat3])

    def _layer(carry, params):
        h, asd, prev = carry
        LW, Lb, b, Wn, Acatn = params
        a1d = _logits_1d(asd)
        gat = _edge_pass(h, a1d, psrc, pdst)[:N]
        hn, asdn, y, o = _mid(gat, prev, LW, Lb, b, Wn, Acatn)
        return (hn, asdn, y), o

    _, os = jax.lax.scan(_layer, (h1, asd1, x), (LWs, Lbs, bs, Wns, Acats))
    return os[2][:, :b3.shape[0]]


# speedup vs baseline: 10.3647x; 1.0055x over previous
import jax
import jax.numpy as jnp
from jax.experimental import pallas as pl
from jax.experimental.pallas import tpu as pltpu
from jax.experimental.pallas import tpu_sc as plsc

N = 10000
E = 320000
D = 128
NPAD = 10240
NW = 32
NT = 16
EPT = E // NW
CAP2 = 448
CH = 64
NCHS = CAP2 // CH
LPAD = NW * CAP2
ROWS = NPAD // NW
TROWS = 336
CHE = 2000
BLK = 1000
GRID = N // BLK

_f32 = jnp.float32
_i32 = jnp.int32

_SCP = pltpu.CompilerParams(use_tc_tiling_on_sc=False,
                            needs_layout_passes=False)
_SCP_EDGE = _SCP


def _bucket(d16):
    return jax.lax.shift_right_logical(
        jax.lax.shift_right_logical(d16, 6) * 6554, 15)


def _vgather(x, idx):
    return jax.lax.gather(
        x, idx[:, None],
        jax.lax.GatherDimensionNumbers(
            offset_dims=(), collapsed_slice_dims=(0,), start_index_map=(0,)),
        (1,), mode=jax.lax.GatherScatterMode.PROMISE_IN_BOUNDS)


def _runs(keys, i16):
    prev = _vgather(keys, jnp.maximum(i16 - 1, 0))
    first = jnp.logical_or(i16 == 0, keys != prev)
    run = plsc.cummax(jnp.where(first, i16, 0))
    nxt = _vgather(keys, jnp.minimum(i16 + 1, 15))
    last = jnp.logical_or(i16 == 15, keys != nxt)
    return run, last


def _part_body(src_hbm, dst_hbm, psrc, pdst,
               es_v, ed_v, os_v, od_v, off_v):
    c = jax.lax.axis_index("c")
    t = jax.lax.axis_index("s")
    wid = c * NT + t
    i16 = jax.lax.iota(_i32, 16)
    z16i = jnp.zeros((16,), _i32)

    off_v[pl.ds(0, 16)] = i16 * CAP2
    off_v[pl.ds(16, 16)] = (i16 + 16) * CAP2

    def _fill(k, carry):
        os_v[pl.ds(k * 16, 16)] = (i16 + k * 16) & 8191
        od_v[pl.ds(k * 16, 16)] = z16i + 2000000
        return carry
    jax.lax.fori_loop(0, LPAD // 16, _fill, 0)

    def _place_chunk(ch, carry):
        pltpu.sync_copy(src_hbm.at[pl.ds(wid * EPT + ch * CHE, CHE)], es_v)
        pltpu.sync_copy(dst_hbm.at[pl.ds(wid * EPT + ch * CHE, CHE)], ed_v)

        def _place(g, carry2):
            s16 = es_v[pl.ds(g * 16, 16)]
            d16 = ed_v[pl.ds(g * 16, 16)]
            bs, perm = plsc.sort_key_val(_bucket(d16), i16)
            run, last = _runs(bs, i16)
            rank = i16 - run
            ss = _vgather(s16, perm)
            ds = _vgather(d16, perm)
            pos = plsc.load_gather(off_v, [bs]) + rank
            plsc.store_scatter(os_v, [pos], ss)
            plsc.store_scatter(od_v, [pos], ds)
            plsc.addupdate_scatter(off_v, [bs], rank + 1, mask=last)
            return carry2
        jax.lax.fori_loop(0, CHE // 16, _place, 0)
        return carry
    jax.lax.fori_loop(0, EPT // CHE, _place_chunk, 0)

    pltpu.sync_copy(os_v.at[pl.ds(0, LPAD)], psrc.at[wid])
    pltpu.sync_copy(od_v.at[pl.ds(0, LPAD)], pdst.at[wid])


def _partition(src, dst):
    mesh = plsc.VectorSubcoreMesh(core_axis_name="c", subcore_axis_name="s")
    fn = pl.kernel(
        _part_body,
        out_type=(jax.ShapeDtypeStruct((NW, LPAD), _i32),
                  jax.ShapeDtypeStruct((NW, LPAD), _i32)),
        mesh=mesh,
        scratch_types=[
            pltpu.VMEM((CHE,), _i32),
            pltpu.VMEM((CHE,), _i32),
            pltpu.VMEM((LPAD + 192,), _i32),
            pltpu.VMEM((LPAD + 192,), _i32),
            pltpu.VMEM((32,), _i32),
        ],
        compiler_params=_SCP,
    )
    return fn(src, dst)


def _logit_body(asdp_hbm, out_hbm, buf_v, out_v):
    c = jax.lax.axis_index("c")
    t = jax.lax.axis_index("s")
    wid = c * NT + t
    i16 = jax.lax.iota(_i32, 16)
    half = jax.lax.shift_right_logical(i16, 1)
    par = jax.lax.bitwise_and(i16, 1)

    for k in range(ROWS // 40):
        pltpu.sync_copy(asdp_hbm.at[pl.ds(wid * ROWS + k * 40, 40)], buf_v)
        for g in range(5):
            v = plsc.load_gather(buf_v, [half + g * 8, par])
            out_v[pl.ds(k * 80 + g * 16, 16)] = v
    pltpu.sync_copy(out_v, out_hbm.at[pl.ds(wid * 2 * ROWS, 2 * ROWS)])


def _logits_1d(asdp):
    mesh = plsc.VectorSubcoreMesh(core_axis_name="c", subcore_axis_name="s")
    fn = pl.kernel(
        _logit_body,
        out_type=jax.ShapeDtypeStruct((2 * NPAD,), _f32),
        mesh=mesh,
        scratch_types=[
            pltpu.VMEM((40, D), _f32),
            pltpu.VMEM((2 * ROWS,), _f32),
        ],
        compiler_params=_SCP,
    )
    return fn(asdp)


def _edge_body(h_hbm, asd1_hbm, psrc, pdst, acc_out,
               src_v, dst_v, dloc_v, rows_v, w_v, tbl_v, s_v, sinv_v, acc_v,
               sem):
    c = jax.lax.axis_index("c")
    t = jax.lax.axis_index("s")
    wid = c * NT + t
    base_row = wid * ROWS
    i16 = jax.lax.iota(_i32, 16)
    z16f = jnp.zeros((16,), _f32)
    z16i = jnp.zeros((16,), _i32)
    ng = D // 16

    pltpu.sync_copy(asd1_hbm, tbl_v)

    def _za(k, carry):
        plsc.store_scatter(acc_v, [z16i + k // ng, i16 + (k % ng) * 16], z16f)
        return carry
    jax.lax.fori_loop(0, TROWS * ng, _za, 0)

    def _zs(k, carry):
        s_v[pl.ds(k * 16, 16)] = z16f
        return carry
    jax.lax.fori_loop(0, TROWS // 16, _zs, 0)

    def _chunk(q, carry):
        w = q // NCHS
        off = wid * CAP2 + (q % NCHS) * CH
        pltpu.sync_copy(psrc.at[w, pl.ds(off, CH)], src_v)
        pltpu.sync_copy(pdst.at[w, pl.ds(off, CH)], dst_v)
        cp = pltpu.async_copy(h_hbm.at[src_v], rows_v, sem)
        for g in range(CH // 16):
            d16 = dst_v[pl.ds(g * 16, 16)]
            s16 = src_v[pl.ds(g * 16, 16)]
            dl16 = jnp.clip(d16 - base_row, 0, ROWS)
            dloc_v[pl.ds(g * 16, 16)] = dl16
            av = plsc.load_gather(tbl_v, [s16 * 2])
            dc = jnp.clip(d16, 0, NPAD - 1)
            dv = plsc.load_gather(tbl_v, [dc * 2 + 1])
            al = av + dv
            al = jnp.where(al >= 0.0, al, al * 0.2)
            wv = jnp.exp(al)
            w_v[pl.ds(g * 16, 16)] = wv
            ks, perm = plsc.sort_key_val(dl16, i16)
            wsrt = _vgather(wv, perm)
            csum = plsc.cumsum(wsrt)
            run, last = _runs(ks, i16)
            pcs = _vgather(csum, jnp.maximum(run - 1, 0))
            pcs = jnp.where(run == 0, 0.0, pcs)
            plsc.addupdate_scatter(s_v, [ks], csum - pcs, mask=last)
        cp.wait()

        def _accum(e, carry3):
            ei = z16i + e
            ws = plsc.load_gather(w_v, [ei])
            dloc = plsc.load_gather(dloc_v, [ei])
            for j in range(ng):
                col = i16 + j * 16
                r = plsc.load_gather(rows_v, [ei, col])
                plsc.addupdate_scatter(acc_v, [dloc, col], r * ws)
            return carry3
        jax.lax.fori_loop(0, CH, _accum, 0, unroll=8)
        return carry
    jax.lax.fori_loop(0, NW * NCHS, _chunk, 0)

    for ck in range(ROWS // CH):
        for g in range(CH // 16):
            n16 = base_row + ck * CH + g * 16 + i16
            nc = jnp.minimum(n16, N - 1)
            src_v[pl.ds(g * 16, 16)] = nc
        cp = pltpu.async_copy(h_hbm.at[src_v], rows_v, sem)
        for g in range(CH // 16):
            n16 = base_row + ck * CH + g * 16 + i16
            nc = jnp.minimum(n16, N - 1)
            av = plsc.load_gather(tbl_v, [nc * 2])
            dv = plsc.load_gather(tbl_v, [nc * 2 + 1])
            al = av + dv
            al = jnp.where(al >= 0.0, al, al * 0.2)
            ws16 = jnp.exp(al)
            w_v[pl.ds(g * 16, 16)] = ws16
            s16 = s_v[pl.ds(ck * CH + g * 16, 16)]
            sinv_v[pl.ds(g * 16, 16)] = 1.0 / (s16 + ws16 + 1e-16)
        cp.wait()

        def _norm(e, carry2):
            ei = z16i + e
            ws = plsc.load_gather(w_v, [ei])
            si = plsc.load_gather(sinv_v, [ei])
            dloc = z16i + ck * CH + e
            for j in range(ng):
                col = i16 + j * 16
                a = plsc.load_gather(acc_v, [dloc, col])
                r = plsc.load_gather(rows_v, [ei, col])
                plsc.store_scatter(acc_v, [dloc, col], (a + ws * r) * si)
            return carry2
        jax.lax.fori_loop(0, CH, _norm, 0)

    for k in range(ROWS // CH):
        pltpu.sync_copy(acc_v.at[pl.ds(k * CH, CH)],
                        acc_out.at[pl.ds(base_row + k * CH, CH)])


def _edge_pass(h, asd1, psrc, pdst):
    mesh = plsc.VectorSubcoreMesh(core_axis_name="c", subcore_axis_name="s")
    fn = pl.kernel(
        _edge_body,
        out_type=jax.ShapeDtypeStruct((NPAD, D), _f32),
        mesh=mesh,
        scratch_types=[
            pltpu.VMEM((CH,), _i32),
            pltpu.VMEM((CH,), _i32),
            pltpu.VMEM((CH,), _i32),
            pltpu.VMEM((CH, D), _f32),
            pltpu.VMEM((CH,), _f32),
            pltpu.VMEM((2 * NPAD,), _f32),
            pltpu.VMEM((TROWS,), _f32),
            pltpu.VMEM((CH,), _f32),
            pltpu.VMEM((TROWS, D), _f32),
            pltpu.SemaphoreType.DMA,
        ],
        compiler_params=_SCP_EDGE,
    )
    return fn(h, asd1, psrc, pdst)


def _asd_pack(asd):
    blk = asd.shape[0]
    return jnp.concatenate([asd, jnp.zeros((blk, D - 2), _f32)], axis=1)


def _pre_body(x_ref, w_ref, a_ref, h_ref, asd_ref):
    h = jnp.dot(x_ref[...], w_ref[...], preferred_element_type=_f32)
    h_ref[...] = h
    asd = jnp.dot(h, a_ref[...], preferred_element_type=_f32)
    asd_ref[...] = _asd_pack(asd)


def _pre(x, W, Acat):
    return pl.pallas_call(
        _pre_body,
        grid=(GRID,),
        in_specs=[pl.BlockSpec((BLK, D), lambda i: (i, 0)),
                  pl.BlockSpec((D, D), lambda i: (0, 0)),
                  pl.BlockSpec((D, 2), lambda i: (0, 0))],
        out_specs=[pl.BlockSpec((BLK, D), lambda i: (i, 0)),
                   pl.BlockSpec((BLK, D), lambda i: (i, 0))],
        out_shape=[jax.ShapeDtypeStruct((N, D), _f32),
                   jax.ShapeDtypeStruct((NPAD, D), _f32)],
    )(x, W, Acat)


def _mid_body(gat_ref, prev_ref, lw_ref, lb_ref, b_ref, wn_ref, an_ref,
              hn_ref, asdn_ref, y_ref, o_ref):
    res = jnp.dot(prev_ref[...], lw_ref[...], preferred_element_type=_f32)
    o = gat_ref[...] + b_ref[...] + res + lb_ref[...]
    o_ref[...] = o
    y = jnp.where(o > 0.0, o, jnp.exp(o) - 1.0)
    y_ref[...] = y
    hn = jnp.dot(y, wn_ref[...], preferred_element_type=_f32)
    hn_ref[...] = hn
    asdn = jnp.dot(hn, an_ref[...], preferred_element_type=_f32)
    asdn_ref[...] = _asd_pack(asdn)


def _mid(gat, prev, LW, Lb, b, Wn, Acatn):
    return pl.pallas_call(
        _mid_body,
        grid=(GRID,),
        in_specs=[pl.BlockSpec((BLK, D), lambda i: (i, 0)),
                  pl.BlockSpec((BLK, D), lambda i: (i, 0)),
                  pl.BlockSpec((D, D), lambda i: (0, 0)),
                  pl.BlockSpec((1, D), lambda i: (0, 0)),
                  pl.BlockSpec((1, D), lambda i: (0, 0)),
                  pl.BlockSpec((D, D), lambda i: (0, 0)),
                  pl.BlockSpec((D, 2), lambda i: (0, 0))],
        out_specs=[pl.BlockSpec((BLK, D), lambda i: (i, 0)),
                   pl.BlockSpec((BLK, D), lambda i: (i, 0)),
                   pl.BlockSpec((BLK, D), lambda i: (i, 0)),
                   pl.BlockSpec((BLK, D), lambda i: (i, 0))],
        out_shape=[jax.ShapeDtypeStruct((N, D), _f32),
                   jax.ShapeDtypeStruct((NPAD, D), _f32),
                   jax.ShapeDtypeStruct((N, D), _f32),
                   jax.ShapeDtypeStruct((N, D), _f32)],
    )(gat, prev, LW, Lb, b, Wn, Acatn)


def _att_cat(att_src, att_dst, pad_to=None):
    a_s = att_src.reshape(-1, 1)
    a_d = att_dst.reshape(-1, 1)
    cat = jnp.concatenate([a_s, a_d], axis=1)
    if pad_to is not None and cat.shape[0] < pad_to:
        cat = jnp.pad(cat, ((0, pad_to - cat.shape[0]), (0, 0)))
    return cat


def kernel(x, edge_index, W1, att_src1, att_dst1, b1, W2, att_src2, att_dst2,
           b2, W3, att_src3, att_dst3, b3, L1W, L1b, L2W, L2b, L3W, L3b):
    src = edge_index[0]
    dst = edge_index[1]

    Acat1 = _att_cat(att_src1, att_dst1)
    Acat2 = _att_cat(att_src2, att_dst2)
    Acat3 = _att_cat(att_src3, att_dst3, pad_to=D)
    W3p = jnp.pad(W3, ((0, 0), (0, D - W3.shape[1])))
    L3Wp = jnp.pad(L3W, ((0, 0), (0, D - L3W.shape[1])))
    pad3 = D - b3.shape[0]
    b3p = jnp.pad(b3, (0, pad3)).reshape(1, D)
    L3bp = jnp.pad(L3b, (0, pad3)).reshape(1, D)

    psrc, pdst = _partition(src, dst)
    h1, asd1 = _pre(x, W1, Acat1)

    LWs = jnp.stack([L1W, L2W, L3Wp])
    Lbs = jnp.stack([L1b.reshape(1, D), L2b.reshape(1, D), L3bp])
    bs = jnp.stack([b1.reshape(1, D), b2.reshape(1, D), b3p])
    Wns = jnp.stack([W2, W3p, W3p])
    Acats = jnp.stack([Acat2, Acat3, Acat3])

    def _layer(carry, params):
        h, asd, prev = carry
        LW, Lb, b, Wn, Acatn = params
        a1d = _logits_1d(asd)
        gat = _edge_pass(h, a1d, psrc, pdst)[:N]
        hn, asdn, y, o = _mid(gat, prev, LW, Lb, b, Wn, Acatn)
        return (hn, asdn, y), o

    _, os = jax.lax.scan(_layer, (h1, asd1, x), (LWs, Lbs, bs, Wns, Acats))
    return os[2][:, :b3.shape[0]]
